# baseline (speedup 1.0000x reference)
import functools

import jax
import jax.numpy as jnp
from jax import lax
from jax.experimental import pallas as pl
from jax.experimental.pallas import tpu as pltpu
from jax.experimental.pallas import tpu_sc as plsc

_NC = 2
_NS = 16
_NW = _NC * _NS
_L = 16

_ROWS = 8
_TILE = _ROWS * 128
_CH = 128


def _render_call(faces, attr128, p2f_2d, bary_2d):
    npix = p2f_2d.shape[0] * 128
    D = 16
    pix_per_w = npix // _NW
    ntiles = pix_per_w // _TILE

    mesh = plsc.VectorSubcoreMesh(core_axis_name="c", subcore_axis_name="s")

    @functools.partial(
        pl.kernel,
        out_type=(
            jax.ShapeDtypeStruct((npix * D // 128, 128), jnp.float32),
            jax.ShapeDtypeStruct((npix // 128, 128), jnp.int32),
        ),
        mesh=mesh,
        scratch_types=[
            pltpu.VMEM((_TILE // 128, 128), jnp.int32),
            pltpu.VMEM((3 * _TILE // 128, 128), jnp.float32),
            pltpu.VMEM((3 * _TILE,), jnp.int32),
            pltpu.VMEM((3 * _TILE,), jnp.int32),
            pltpu.VMEM((3 * _CH, 128), jnp.float32),
            pltpu.VMEM((_TILE * D // 128, 128), jnp.float32),
            pltpu.VMEM((_TILE // 128, 128), jnp.int32),
            pltpu.SemaphoreType.DMA,
        ],
    )
    def render(faces_h, attr_h, p2f_h, bary_h, out_h, mask_h,
               p2f_v, bary_v, fidx_v, vert_v, attr_v, out_v, mask_v, sem):
        wid = lax.axis_index("s") * _NC + lax.axis_index("c")
        base_pix = wid * pix_per_w

        def tile_body(t, carry):
            p0 = pl.multiple_of(base_pix + t * _TILE, _TILE)
            pltpu.sync_copy(p2f_h.at[pl.ds(pl.multiple_of(p0 // 128, 8), _TILE // 128)], p2f_v)
            pltpu.sync_copy(bary_h.at[pl.ds(pl.multiple_of(3 * p0 // 128, 8), 3 * _TILE // 128)],
                            bary_v)

            for j in range(_ROWS):
                for g in range(128 // _L):
                    off = j * 128 + g * _L
                    f3 = p2f_v[j, pl.ds(g * _L, _L)] * 3
                    fidx_v[pl.ds(0 * _TILE + off, _L)] = f3
                    fidx_v[pl.ds(1 * _TILE + off, _L)] = f3 + 1
                    fidx_v[pl.ds(2 * _TILE + off, _L)] = f3 + 2

            fcps = [pltpu.async_copy(
                        faces_h.at[fidx_v.at[pl.ds(k * _TILE, _TILE)]],
                        vert_v.at[pl.ds(k * _TILE, _TILE)], sem)
                    for k in range(3)]
            for cp in fcps:
                cp.wait()

            for c in range(_TILE // _CH):
                acps = [pltpu.async_copy(
                            attr_h.at[vert_v.at[pl.ds(k * _TILE + c * _CH,
                                                      _CH)]],
                            attr_v.at[pl.ds(k * _CH, _CH)], sem)
                        for k in range(3)]
                for cp in acps:
                    cp.wait()

                def g_body(g, cc, c=c):
                    base = g * _L
                    wbase = c * 3 * 128 + g * (3 * _L)
                    ch = [bary_v[(wbase + i * _L) // 128,
                                 pl.ds((wbase + i * _L) % 128, _L)]
                          for i in range(3)]
                    for l in range(_L):
                        p = c * 128 + base + l
                        q = 3 * l
                        w = [ch[(q + i) // _L][(q + i) % _L] for i in range(3)]
                        acc = (attr_v[0 * _CH + base + l, pl.ds(0, _L)] * w[0]
                               + attr_v[1 * _CH + base + l, pl.ds(0, _L)]
                               * w[1]
                               + attr_v[2 * _CH + base + l, pl.ds(0, _L)]
                               * w[2])
                        out_v[p // 8, pl.ds((p % 8) * D, D)] = acc
                    return cc
                lax.fori_loop(0, _CH // _L, g_body, 0)

            one = jnp.full((_L,), 1, jnp.int32)
            zero = jnp.full((_L,), 0, jnp.int32)
            for j in range(_ROWS):
                for g in range(128 // _L):
                    sl = pl.ds(g * _L, _L)
                    v = p2f_v[j, sl]
                    mask_v[j, sl] = jnp.where(v != -1, one, zero)

            pltpu.sync_copy(out_v,
                            out_h.at[pl.ds(pl.multiple_of(p0 * D // 128, 8), _TILE * D // 128)])
            pltpu.sync_copy(mask_v,
                            mask_h.at[pl.ds(pl.multiple_of(p0 // 128, 8), _TILE // 128)])
            return carry

        lax.fori_loop(0, ntiles, tile_body, 0)

    return render(faces, attr128, p2f_2d, bary_2d)


def kernel(vertices, faces, attributes, pix_to_face, bary_coords):
    H, W = pix_to_face.shape
    N = H * W
    D = attributes.shape[1]
    p2f_2d = pix_to_face.reshape(N // 128, 128)
    bary_2d = bary_coords.reshape(N * 3 // 128, 128)
    faces_flat = faces.reshape(faces.shape[0] * 3)
    attr128 = jnp.pad(attributes, ((0, 0), (0, 128 - D)))
    out, mask_i = _render_call(faces_flat, attr128, p2f_2d, bary_2d)
    attribute_map = out.reshape(H, W, D)
    mask = mask_i.reshape(H, W).astype(bool)
    return (attribute_map, mask)


# speedup vs baseline: 1.6419x; 1.6419x over previous
import functools

import jax
import jax.numpy as jnp
from jax import lax
from jax.experimental import pallas as pl
from jax.experimental.pallas import tpu as pltpu
from jax.experimental.pallas import tpu_sc as plsc

_NC = 2
_NS = 16
_NW = _NC * _NS
_L = 16

_ROWS = 8
_TILE = _ROWS * 128


def _render_call(faces, attributes, p2f_2d, bary_2d):
    npix = p2f_2d.shape[0]
    D = attributes.shape[1]
    pix_per_w = npix // _NW
    ntiles = pix_per_w // _TILE

    mesh = plsc.VectorSubcoreMesh(core_axis_name="c", subcore_axis_name="s")

    @functools.partial(
        pl.kernel,
        out_type=(
            jax.ShapeDtypeStruct((npix * D,), jnp.float32),
            jax.ShapeDtypeStruct((npix,), jnp.int32),
        ),
        mesh=mesh,
        compiler_params=pltpu.CompilerParams(use_tc_tiling_on_sc=False),
        scratch_types=[
            pltpu.VMEM((_TILE,), jnp.int32),
            pltpu.VMEM((3, _TILE), jnp.float32),
            pltpu.VMEM((3 * _TILE,), jnp.int32),
            pltpu.VMEM((3 * _TILE,), jnp.int32),
            pltpu.VMEM((3 * _TILE, D), jnp.float32),
            pltpu.VMEM((_TILE * D,), jnp.float32),
            pltpu.VMEM((_TILE,), jnp.int32),
            pltpu.SemaphoreType.DMA,
        ],
    )
    def render(faces_h, attr_h, p2f_h, bary_h, out_h, mask_h,
               p2f_v, bary_v, fidx_v, vert_v, attr_v, out_v, mask_v, sem):
        wid = lax.axis_index("s") * _NC + lax.axis_index("c")
        base_pix = wid * pix_per_w

        def tile_body(t, carry):
            p0 = base_pix + t * _TILE
            pltpu.sync_copy(p2f_h.at[pl.ds(p0, _TILE)], p2f_v)
            for k in range(3):
                pltpu.sync_copy(bary_h.at[k, pl.ds(p0, _TILE)],
                                bary_v.at[k])

            for j in range(_ROWS):
                for g in range(128 // _L):
                    off = j * 128 + g * _L
                    f3 = p2f_v[pl.ds(off, _L)] * 3
                    fidx_v[pl.ds(0 * _TILE + off, _L)] = f3
                    fidx_v[pl.ds(1 * _TILE + off, _L)] = f3 + 1
                    fidx_v[pl.ds(2 * _TILE + off, _L)] = f3 + 2

            fcps = [pltpu.async_copy(
                        faces_h.at[fidx_v.at[pl.ds(k * _TILE, _TILE)]],
                        vert_v.at[pl.ds(k * _TILE, _TILE)], sem)
                    for k in range(3)]
            acps = []
            for k in range(3):
                fcps[k].wait()
                acps.append(pltpu.async_copy(
                    attr_h.at[vert_v.at[pl.ds(k * _TILE, _TILE)]],
                    attr_v.at[pl.ds(k * _TILE, _TILE)], sem))
            for cp in acps:
                cp.wait()

            for j in range(_ROWS):
                def g_body(g, c, j=j):
                    base = g * _L
                    ch = [bary_v[i, pl.ds(j * 128 + base, _L)]
                          for i in range(3)]
                    for l in range(_L):
                        p = j * 128 + base + l
                        acc = (attr_v[0 * _TILE + p, :] * ch[0][l]
                               + attr_v[1 * _TILE + p, :] * ch[1][l]
                               + attr_v[2 * _TILE + p, :] * ch[2][l])
                        out_v[pl.ds(p * D, D)] = acc
                    return c
                lax.fori_loop(0, 128 // _L, g_body, 0)

            one = jnp.full((_L,), 1, jnp.int32)
            zero = jnp.full((_L,), 0, jnp.int32)
            for j in range(_ROWS):
                for g in range(128 // _L):
                    sl = pl.ds(j * 128 + g * _L, _L)
                    v = p2f_v[sl]
                    mask_v[sl] = jnp.where(v != -1, one, zero)

            pltpu.sync_copy(out_v, out_h.at[pl.ds(p0 * D, _TILE * D)])
            pltpu.sync_copy(mask_v, mask_h.at[pl.ds(p0, _TILE)])
            return carry

        lax.fori_loop(0, ntiles, tile_body, 0)

    return render(faces, attributes, p2f_2d, bary_2d)


def kernel(vertices, faces, attributes, pix_to_face, bary_coords):
    H, W = pix_to_face.shape
    N = H * W
    D = attributes.shape[1]
    p2f_2d = pix_to_face.reshape(N)
    bary_2d = bary_coords.reshape(N, 3).T
    faces_flat = faces.reshape(faces.shape[0] * 3)
    out, mask_i = _render_call(faces_flat, attributes, p2f_2d, bary_2d)
    attribute_map = out.reshape(H, W, D)
    mask = mask_i.reshape(H, W).astype(bool)
    return (attribute_map, mask)
